# baseline (speedup 1.0000x reference)
import jax
import jax.numpy as jnp
from jax.experimental import pallas as pl
import jax.experimental.pallas.tpu as pltpu

N = 4096
H1 = 128
H2 = 64
D_IN = 256

BM = 1024
BN = 1024
NI = N // BM
NJ = N // BN
C = 7


def _is_cached(lin):
    return jnp.logical_and(lin % 2 == 1, lin <= 2 * C - 1)


def _fused_kernel(adj_ref, y_ref, w1_ref, w2_ref, scale_ref, beta_ref,
                  sc2_ref, b2_ref, feat_ref, struct_ref,
                  t_ref, u_ref, s1_ref, acc_ref, cache_ref, copy_sem):
    p = pl.program_id(0)
    i = pl.program_id(1)
    j = pl.program_id(2)
    lin = i * NJ + j
    slot = (lin - 1) // 2

    @pl.when(jnp.logical_and(p == 0, lin == 0))
    def _():
        t_ref[...] = jnp.dot(y_ref[...], w1_ref[...],
                             preferred_element_type=jnp.float32)

    @pl.when(j == 0)
    def _():
        acc_ref[...] = jnp.zeros_like(acc_ref)

    @pl.when(p == 0)
    def _():
        acc_ref[...] += jnp.dot(adj_ref[...], t_ref[pl.ds(j * BN, BN), :],
                                preferred_element_type=jnp.float32)

        @pl.when(_is_cached(lin))
        def _():
            cp = pltpu.make_async_copy(
                adj_ref, cache_ref.at[pl.ds(slot * BM, BM), :], copy_sem)
            cp.start()
            cp.wait()

        @pl.when(j == NJ - 1)
        def _():
            hs = (jnp.maximum(acc_ref[...], 0.0) * scale_ref[...]
                  + beta_ref[...])
            s1_ref[pl.ds(i * BM, BM), :] = hs[:, H1:].astype(jnp.bfloat16)
            u_ref[pl.ds(i * BM, BM), :] = jnp.dot(
                hs[:, :H1], w2_ref[...],
                preferred_element_type=jnp.float32).astype(jnp.bfloat16)

    @pl.when(p == 1)
    def _():
        u_j = u_ref[pl.ds(j * BN, BN), :].astype(jnp.float32)

        @pl.when(_is_cached(lin))
        def _():
            acc_ref[...] += jnp.dot(cache_ref[pl.ds(slot * BM, BM), :], u_j,
                                    preferred_element_type=jnp.float32)

        @pl.when(jnp.logical_not(_is_cached(lin)))
        def _():
            acc_ref[...] += jnp.dot(adj_ref[...], u_j,
                                    preferred_element_type=jnp.float32)

        struct_ref[...] = jax.lax.dot_general(
            s1_ref[pl.ds(i * BM, BM), :], s1_ref[pl.ds(j * BN, BN), :],
            (((1,), (1,)), ((), ())), preferred_element_type=jnp.float32)

        @pl.when(j == NJ - 1)
        def _():
            feat_ref[...] = (jnp.maximum(acc_ref[...], 0.0) * sc2_ref[...]
                             + b2_ref[...])


def _adj_index_map(p, i, j):
    lin = i * NJ + j
    cached = jnp.logical_and(p == 1, _is_cached(lin))
    prev = lin - 1
    return (jnp.where(cached, prev // NJ, i), jnp.where(cached, prev % NJ, j))


def kernel(y_features, adj, W_fd1, W_fd2, W_sd1, g1, b1, g2, b2, g3, b3):
    inv = 1.0 / jnp.sqrt(jnp.float32(1.0 + 1e-5))
    w_cat = jnp.concatenate([W_fd1, W_sd1], axis=1)
    scale_cat = (jnp.concatenate([g1, g3]) * inv).reshape(1, 2 * H1)
    beta_cat = jnp.concatenate([b1, b3]).reshape(1, 2 * H1)
    sc2 = (g2 * inv).reshape(1, D_IN)
    b2r = b2.reshape(1, D_IN)

    feat, struct = pl.pallas_call(
        _fused_kernel,
        grid=(2, NI, NJ),
        in_specs=[
            pl.BlockSpec((BM, BN), _adj_index_map),
            pl.BlockSpec((N, H2), lambda p, i, j: (0, 0)),
            pl.BlockSpec((H2, 2 * H1), lambda p, i, j: (0, 0)),
            pl.BlockSpec((H1, D_IN), lambda p, i, j: (0, 0)),
            pl.BlockSpec((1, 2 * H1), lambda p, i, j: (0, 0)),
            pl.BlockSpec((1, 2 * H1), lambda p, i, j: (0, 0)),
            pl.BlockSpec((1, D_IN), lambda p, i, j: (0, 0)),
            pl.BlockSpec((1, D_IN), lambda p, i, j: (0, 0)),
        ],
        out_specs=[
            pl.BlockSpec((BM, D_IN),
                         lambda p, i, j: (jnp.where(p == 1, i, 0), 0)),
            pl.BlockSpec((BM, BN),
                         lambda p, i, j: (jnp.where(p == 1, i, 0),
                                          jnp.where(p == 1, j, 0))),
        ],
        out_shape=[
            jax.ShapeDtypeStruct((N, D_IN), jnp.float32),
            jax.ShapeDtypeStruct((N, N), jnp.float32),
        ],
        scratch_shapes=[
            pltpu.VMEM((N, D_IN), jnp.float32),
            pltpu.VMEM((N, D_IN), jnp.bfloat16),
            pltpu.VMEM((N, H1), jnp.bfloat16),
            pltpu.VMEM((BM, D_IN), jnp.float32),
            pltpu.VMEM((C * BM, BN), jnp.float32),
            pltpu.SemaphoreType.DMA,
        ],
    )(adj, y_features, w_cat, W_fd2, scale_cat, beta_cat, sc2, b2r)

    return (feat, struct)
